# Initial kernel scaffold; baseline (speedup 1.0000x reference)
import functools

import jax
import jax.numpy as jnp
from jax import lax
from jax.experimental import pallas as pl
from jax.experimental.pallas import tpu as pltpu
from jax.experimental.pallas import tpu_sc as plsc

N = 10000
NPAD = 10240
PAD_ROWS = NPAD - N
E = 320000
D_IN = 128
D_H = 64
NC = 2
NS = 16
NW = NC * NS
CHUNK = 128
CPT = 80
ETP = NW * CPT * CHUNK
IDX_ROWS = ETP // CHUNK
ACC_TILE = NPAD // NS

_SC_MESH = plsc.VectorSubcoreMesh(
    core_axis_name="c", subcore_axis_name="s", num_cores=NC, num_subcores=NS)


def _fill_rows(buf, rows, val):
  cols = buf.shape[1]
  v = jnp.full((16,), val, jnp.float32)

  @pl.loop(0, rows)
  def _(i):
    for c in range(cols // 16):
      buf[i, pl.ds(c * 16, 16)] = v


@functools.partial(
    pl.kernel,
    out_type=jax.ShapeDtypeStruct((NC * NPAD, 16), jnp.float32),
    mesh=_SC_MESH,
    scratch_types=[
        pltpu.VMEM((CPT, CHUNK), jnp.int32),
        pltpu.VMEM((CHUNK, 16), jnp.float32),
        pltpu.VMEM((CHUNK, 16), jnp.float32),
        pltpu.VMEM_SHARED((NPAD, 16), jnp.float32),
    ],
)
def _deg_kernel(dst_hbm, out_hbm, dstbuf, ones, zeros, acc):
  cid = lax.axis_index("c")
  sid = lax.axis_index("s")
  wid = sid * NC + cid

  _fill_rows(ones, CHUNK, 1.0)
  _fill_rows(zeros, CHUNK, 0.0)
  base = sid * ACC_TILE
  for r in range(ACC_TILE // CHUNK):
    pltpu.sync_copy(zeros, acc.at[pl.ds(base + r * CHUNK, CHUNK)])
  plsc.subcore_barrier()

  pltpu.sync_copy(dst_hbm.at[pl.ds(wid * CPT, CPT)], dstbuf)

  @pl.loop(0, CPT)
  def _(j):
    pltpu.sync_copy(ones, acc.at[dstbuf.at[j]], add=True)

  plsc.subcore_barrier()
  pltpu.sync_copy(acc.at[pl.ds(base, ACC_TILE)],
                  out_hbm.at[pl.ds(cid * NPAD + base, ACC_TILE)])


@functools.partial(
    pl.kernel,
    out_type=jax.ShapeDtypeStruct((NC * NPAD, D_H), jnp.float32),
    mesh=_SC_MESH,
    scratch_types=[
        pltpu.VMEM((CPT, CHUNK), jnp.int32),
        pltpu.VMEM((CPT, CHUNK), jnp.int32),
        pltpu.VMEM((CHUNK, D_H), jnp.float32),
        pltpu.VMEM((CHUNK, D_H), jnp.float32),
        pltpu.VMEM((CHUNK, D_H), jnp.float32),
        pltpu.VMEM_SHARED((NPAD, D_H), jnp.float32),
        pltpu.SemaphoreType.DMA,
        pltpu.SemaphoreType.DMA,
    ],
)
def _agg_kernel(src_hbm, dst_hbm, h_hbm, out_hbm,
                srcbuf, dstbuf, rows_a, rows_b, zeros, acc, sem_a, sem_b):
  cid = lax.axis_index("c")
  sid = lax.axis_index("s")
  wid = sid * NC + cid

  _fill_rows(zeros, CHUNK, 0.0)
  base = sid * ACC_TILE
  for r in range(ACC_TILE // CHUNK):
    pltpu.sync_copy(zeros, acc.at[pl.ds(base + r * CHUNK, CHUNK)])
  plsc.subcore_barrier()

  pltpu.sync_copy(src_hbm.at[pl.ds(wid * CPT, CPT)], srcbuf)
  pltpu.sync_copy(dst_hbm.at[pl.ds(wid * CPT, CPT)], dstbuf)

  pltpu.async_copy(h_hbm.at[srcbuf.at[0]], rows_a, sem_a)
  pltpu.async_copy(h_hbm.at[srcbuf.at[1]], rows_b, sem_b)

  @pl.loop(0, CPT - 2, step=2)
  def _(j):
    pltpu.make_async_copy(h_hbm.at[srcbuf.at[0]], rows_a, sem_a).wait()
    pltpu.sync_copy(rows_a, acc.at[dstbuf.at[j]], add=True)
    pltpu.async_copy(h_hbm.at[srcbuf.at[j + 2]], rows_a, sem_a)
    pltpu.make_async_copy(h_hbm.at[srcbuf.at[0]], rows_b, sem_b).wait()
    pltpu.sync_copy(rows_b, acc.at[dstbuf.at[j + 1]], add=True)
    pltpu.async_copy(h_hbm.at[srcbuf.at[j + 3]], rows_b, sem_b)

  pltpu.make_async_copy(h_hbm.at[srcbuf.at[0]], rows_a, sem_a).wait()
  pltpu.sync_copy(rows_a, acc.at[dstbuf.at[CPT - 2]], add=True)
  pltpu.make_async_copy(h_hbm.at[srcbuf.at[0]], rows_b, sem_b).wait()
  pltpu.sync_copy(rows_b, acc.at[dstbuf.at[CPT - 1]], add=True)

  plsc.subcore_barrier()
  pltpu.sync_copy(acc.at[pl.ds(base, ACC_TILE)],
                  out_hbm.at[pl.ds(cid * NPAD + base, ACC_TILE)])


R = 2048
GRID = NPAD // R
DSPEC = [
    pl.BlockSpec((R, 16), lambda i: (i, 0)),
    pl.BlockSpec((R, 16), lambda i: (i + GRID, 0)),
]


def _dinv(dega, degb):
  return lax.rsqrt(dega[:, 0:1] + degb[:, 0:1] + 1.0)


def _k1_body(x_ref, w_ref, da_ref, db_ref, o_ref):
  h = jnp.dot(x_ref[...], w_ref[...], preferred_element_type=jnp.float32)
  o_ref[...] = h * _dinv(da_ref[...], db_ref[...])


def _k1(xp, W1, deg2):
  return pl.pallas_call(
      _k1_body,
      grid=(GRID,),
      in_specs=[
          pl.BlockSpec((R, D_IN), lambda i: (i, 0)),
          pl.BlockSpec((D_IN, D_H), lambda i: (0, 0)),
          *DSPEC,
      ],
      out_specs=pl.BlockSpec((R, D_H), lambda i: (i, 0)),
      out_shape=jax.ShapeDtypeStruct((NPAD, D_H), jnp.float32),
  )(xp, W1, deg2, deg2)


def _k2_body(s1_ref, s2_ref, hp_ref, da_ref, db_ref, b_ref, w_ref, o_ref):
  dinv = _dinv(da_ref[...], db_ref[...])
  t = (s1_ref[...] + s2_ref[...] + hp_ref[...]) * dinv
  z = jnp.maximum(t + b_ref[...], 0.0)
  o_ref[...] = jnp.dot(z, w_ref[...], preferred_element_type=jnp.float32) * dinv


def _k2(s, hp, deg2, b, W):
  return pl.pallas_call(
      _k2_body,
      grid=(GRID,),
      in_specs=[
          pl.BlockSpec((R, D_H), lambda i: (i, 0)),
          pl.BlockSpec((R, D_H), lambda i: (i + GRID, 0)),
          pl.BlockSpec((R, D_H), lambda i: (i, 0)),
          *DSPEC,
          pl.BlockSpec((1, D_H), lambda i: (0, 0)),
          pl.BlockSpec((D_H, D_H), lambda i: (0, 0)),
      ],
      out_specs=pl.BlockSpec((R, D_H), lambda i: (i, 0)),
      out_shape=jax.ShapeDtypeStruct((NPAD, D_H), jnp.float32),
  )(s, s, hp, deg2, deg2, b, W)


def _k2b_body(s1_ref, s2_ref, hp_ref, da_ref, db_ref, b_ref, o_ref):
  dinv = _dinv(da_ref[...], db_ref[...])
  t = (s1_ref[...] + s2_ref[...] + hp_ref[...]) * dinv
  o_ref[...] = jnp.maximum(t + b_ref[...], 0.0) * dinv


def _k2b(s, hp, deg2, b):
  return pl.pallas_call(
      _k2b_body,
      grid=(GRID,),
      in_specs=[
          pl.BlockSpec((R, D_H), lambda i: (i, 0)),
          pl.BlockSpec((R, D_H), lambda i: (i + GRID, 0)),
          pl.BlockSpec((R, D_H), lambda i: (i, 0)),
          *DSPEC,
          pl.BlockSpec((1, D_H), lambda i: (0, 0)),
      ],
      out_specs=pl.BlockSpec((R, D_H), lambda i: (i, 0)),
      out_shape=jax.ShapeDtypeStruct((NPAD, D_H), jnp.float32),
  )(s, s, hp, deg2, deg2, b)


def _k3_body(s1_ref, s2_ref, hp_ref, da_ref, db_ref, b_ref, w_ref, o_ref):
  dinv = _dinv(da_ref[...], db_ref[...])
  t = (s1_ref[...] + s2_ref[...] + hp_ref[...]) * dinv
  o_ref[...] = (jnp.dot(t, w_ref[...], preferred_element_type=jnp.float32)
                + b_ref[...])


def _k3(s, hp, deg2, b, W):
  return pl.pallas_call(
      _k3_body,
      grid=(GRID,),
      in_specs=[
          pl.BlockSpec((R, D_H), lambda i: (i, 0)),
          pl.BlockSpec((R, D_H), lambda i: (i + GRID, 0)),
          pl.BlockSpec((R, D_H), lambda i: (i, 0)),
          *DSPEC,
          pl.BlockSpec((1, D_IN), lambda i: (0, 0)),
          pl.BlockSpec((D_H, D_IN), lambda i: (0, 0)),
      ],
      out_specs=pl.BlockSpec((R, D_IN), lambda i: (i, 0)),
      out_shape=jax.ShapeDtypeStruct((NPAD, D_IN), jnp.float32),
  )(s, s, hp, deg2, deg2, b, W)


def kernel(x, edge_index, is_test, W1, b1, W2, b2, W3, b3):
  del is_test

  pad_idx = N + (jnp.arange(ETP - E, dtype=jnp.int32) % PAD_ROWS)
  srcp = jnp.concatenate(
      [edge_index[0].astype(jnp.int32), pad_idx]).reshape(IDX_ROWS, CHUNK)
  dstp = jnp.concatenate(
      [edge_index[1].astype(jnp.int32), pad_idx]).reshape(IDX_ROWS, CHUNK)
  xp = jnp.pad(x, ((0, NPAD - N), (0, 0)))
  b1r = b1.reshape(1, D_H)
  b2r = b2.reshape(1, D_H)
  b3r = b3.reshape(1, D_IN)

  deg2 = _deg_kernel(dstp)
  h1p = _k1(xp, W1, deg2)
  s1 = _agg_kernel(srcp, dstp, h1p)
  h2p = _k2(s1, h1p, deg2, b1r, W2)
  s2 = _agg_kernel(srcp, dstp, h2p)
  h3p = _k2b(s2, h2p, deg2, b2r)
  s3 = _agg_kernel(srcp, dstp, h3p)
  out = _k3(s3, h3p, deg2, b3r, W3)
  return out[:N]


# speedup vs baseline: 35.0326x; 35.0326x over previous
import functools

import jax
import jax.numpy as jnp
from jax import lax
from jax.experimental import pallas as pl
from jax.experimental.pallas import tpu as pltpu
from jax.experimental.pallas import tpu_sc as plsc

N = 10000
NPAD = 10240
PAD_ROWS = NPAD - N
E = 320000
D_IN = 128
D_H = 64
NC = 2
NS = 16
NW = NC * NS
CHUNK = 128
CPT = 80
ETP = NW * CPT * CHUNK
IDX_ROWS = ETP // CHUNK
ACC_TILE = NPAD // NS

_SC_MESH = plsc.VectorSubcoreMesh(
    core_axis_name="c", subcore_axis_name="s", num_cores=NC, num_subcores=NS)
_SC_PARAMS = pltpu.CompilerParams(use_tc_tiling_on_sc=False)


def _fill_rows(buf, rows, val):
  cols = buf.shape[1]
  v = jnp.full((16,), val, jnp.float32)

  @pl.loop(0, rows)
  def _(i):
    for c in range(cols // 16):
      buf[i, pl.ds(c * 16, 16)] = v


@functools.partial(
    pl.kernel,
    out_type=jax.ShapeDtypeStruct((NC * NPAD, 16), jnp.float32),
    mesh=_SC_MESH,
    compiler_params=_SC_PARAMS,
    scratch_types=[
        pltpu.VMEM((CPT, CHUNK), jnp.int32),
        pltpu.VMEM((CHUNK, 16), jnp.float32),
        pltpu.VMEM((CHUNK, 16), jnp.float32),
        pltpu.VMEM_SHARED((NPAD, 16), jnp.float32),
    ],
)
def _deg_kernel(dst_hbm, out_hbm, dstbuf, ones, zeros, acc):
  cid = lax.axis_index("c")
  sid = lax.axis_index("s")
  wid = sid * NC + cid

  _fill_rows(ones, CHUNK, 1.0)
  _fill_rows(zeros, CHUNK, 0.0)
  base = sid * ACC_TILE
  for r in range(ACC_TILE // CHUNK):
    pltpu.sync_copy(zeros, acc.at[pl.ds(base + r * CHUNK, CHUNK)])
  plsc.subcore_barrier()

  pltpu.sync_copy(dst_hbm.at[pl.ds(wid * CPT, CPT)], dstbuf)

  @pl.loop(0, CPT)
  def _(j):
    pltpu.sync_copy(ones, acc.at[dstbuf.at[j]], add=True)

  plsc.subcore_barrier()
  pltpu.sync_copy(acc.at[pl.ds(base, ACC_TILE)],
                  out_hbm.at[pl.ds(cid * NPAD + base, ACC_TILE)])


@functools.partial(
    pl.kernel,
    out_type=jax.ShapeDtypeStruct((NC * NPAD, D_H), jnp.float32),
    mesh=_SC_MESH,
    compiler_params=_SC_PARAMS,
    scratch_types=[
        pltpu.VMEM((CPT, CHUNK), jnp.int32),
        pltpu.VMEM((CPT, CHUNK), jnp.int32),
        pltpu.VMEM((CHUNK, D_H), jnp.float32),
        pltpu.VMEM((CHUNK, D_H), jnp.float32),
        pltpu.VMEM((CHUNK, D_H), jnp.float32),
        pltpu.VMEM_SHARED((NPAD, D_H), jnp.float32),
        pltpu.SemaphoreType.DMA,
        pltpu.SemaphoreType.DMA,
    ],
)
def _agg_kernel(src_hbm, dst_hbm, h_hbm, out_hbm,
                srcbuf, dstbuf, rows_a, rows_b, zeros, acc, sem_a, sem_b):
  cid = lax.axis_index("c")
  sid = lax.axis_index("s")
  wid = sid * NC + cid

  _fill_rows(zeros, CHUNK, 0.0)
  base = sid * ACC_TILE
  for r in range(ACC_TILE // CHUNK):
    pltpu.sync_copy(zeros, acc.at[pl.ds(base + r * CHUNK, CHUNK)])
  plsc.subcore_barrier()

  pltpu.sync_copy(src_hbm.at[pl.ds(wid * CPT, CPT)], srcbuf)
  pltpu.sync_copy(dst_hbm.at[pl.ds(wid * CPT, CPT)], dstbuf)

  pltpu.async_copy(h_hbm.at[srcbuf.at[0]], rows_a, sem_a)
  pltpu.async_copy(h_hbm.at[srcbuf.at[1]], rows_b, sem_b)

  @pl.loop(0, CPT - 2, step=2)
  def _(j):
    pltpu.make_async_copy(h_hbm.at[srcbuf.at[0]], rows_a, sem_a).wait()
    pltpu.sync_copy(rows_a, acc.at[dstbuf.at[j]], add=True)
    pltpu.async_copy(h_hbm.at[srcbuf.at[j + 2]], rows_a, sem_a)
    pltpu.make_async_copy(h_hbm.at[srcbuf.at[0]], rows_b, sem_b).wait()
    pltpu.sync_copy(rows_b, acc.at[dstbuf.at[j + 1]], add=True)
    pltpu.async_copy(h_hbm.at[srcbuf.at[j + 3]], rows_b, sem_b)

  pltpu.make_async_copy(h_hbm.at[srcbuf.at[0]], rows_a, sem_a).wait()
  pltpu.sync_copy(rows_a, acc.at[dstbuf.at[CPT - 2]], add=True)
  pltpu.make_async_copy(h_hbm.at[srcbuf.at[0]], rows_b, sem_b).wait()
  pltpu.sync_copy(rows_b, acc.at[dstbuf.at[CPT - 1]], add=True)

  plsc.subcore_barrier()
  pltpu.sync_copy(acc.at[pl.ds(base, ACC_TILE)],
                  out_hbm.at[pl.ds(cid * NPAD + base, ACC_TILE)])


R = 2048
GRID = NPAD // R
DSPEC = [
    pl.BlockSpec((R, 16), lambda i: (i, 0)),
    pl.BlockSpec((R, 16), lambda i: (i + GRID, 0)),
]


def _dinv(dega, degb):
  return lax.rsqrt(dega[:, 0:1] + degb[:, 0:1] + 1.0)


def _k1_body(x_ref, w_ref, da_ref, db_ref, o_ref):
  h = jnp.dot(x_ref[...], w_ref[...], preferred_element_type=jnp.float32)
  o_ref[...] = h * _dinv(da_ref[...], db_ref[...])


def _k1(xp, W1, deg2):
  return pl.pallas_call(
      _k1_body,
      grid=(GRID,),
      in_specs=[
          pl.BlockSpec((R, D_IN), lambda i: (i, 0)),
          pl.BlockSpec((D_IN, D_H), lambda i: (0, 0)),
          *DSPEC,
      ],
      out_specs=pl.BlockSpec((R, D_H), lambda i: (i, 0)),
      out_shape=jax.ShapeDtypeStruct((NPAD, D_H), jnp.float32),
  )(xp, W1, deg2, deg2)


def _k2_body(s1_ref, s2_ref, hp_ref, da_ref, db_ref, b_ref, w_ref, o_ref):
  dinv = _dinv(da_ref[...], db_ref[...])
  t = (s1_ref[...] + s2_ref[...] + hp_ref[...]) * dinv
  z = jnp.maximum(t + b_ref[...], 0.0)
  o_ref[...] = jnp.dot(z, w_ref[...], preferred_element_type=jnp.float32) * dinv


def _k2(s, hp, deg2, b, W):
  return pl.pallas_call(
      _k2_body,
      grid=(GRID,),
      in_specs=[
          pl.BlockSpec((R, D_H), lambda i: (i, 0)),
          pl.BlockSpec((R, D_H), lambda i: (i + GRID, 0)),
          pl.BlockSpec((R, D_H), lambda i: (i, 0)),
          *DSPEC,
          pl.BlockSpec((1, D_H), lambda i: (0, 0)),
          pl.BlockSpec((D_H, D_H), lambda i: (0, 0)),
      ],
      out_specs=pl.BlockSpec((R, D_H), lambda i: (i, 0)),
      out_shape=jax.ShapeDtypeStruct((NPAD, D_H), jnp.float32),
  )(s, s, hp, deg2, deg2, b, W)


def _k2b_body(s1_ref, s2_ref, hp_ref, da_ref, db_ref, b_ref, o_ref):
  dinv = _dinv(da_ref[...], db_ref[...])
  t = (s1_ref[...] + s2_ref[...] + hp_ref[...]) * dinv
  o_ref[...] = jnp.maximum(t + b_ref[...], 0.0) * dinv


def _k2b(s, hp, deg2, b):
  return pl.pallas_call(
      _k2b_body,
      grid=(GRID,),
      in_specs=[
          pl.BlockSpec((R, D_H), lambda i: (i, 0)),
          pl.BlockSpec((R, D_H), lambda i: (i + GRID, 0)),
          pl.BlockSpec((R, D_H), lambda i: (i, 0)),
          *DSPEC,
          pl.BlockSpec((1, D_H), lambda i: (0, 0)),
      ],
      out_specs=pl.BlockSpec((R, D_H), lambda i: (i, 0)),
      out_shape=jax.ShapeDtypeStruct((NPAD, D_H), jnp.float32),
  )(s, s, hp, deg2, deg2, b)


def _k3_body(s1_ref, s2_ref, hp_ref, da_ref, db_ref, b_ref, w_ref, o_ref):
  dinv = _dinv(da_ref[...], db_ref[...])
  t = (s1_ref[...] + s2_ref[...] + hp_ref[...]) * dinv
  o_ref[...] = (jnp.dot(t, w_ref[...], preferred_element_type=jnp.float32)
                + b_ref[...])


def _k3(s, hp, deg2, b, W):
  return pl.pallas_call(
      _k3_body,
      grid=(GRID,),
      in_specs=[
          pl.BlockSpec((R, D_H), lambda i: (i, 0)),
          pl.BlockSpec((R, D_H), lambda i: (i + GRID, 0)),
          pl.BlockSpec((R, D_H), lambda i: (i, 0)),
          *DSPEC,
          pl.BlockSpec((1, D_IN), lambda i: (0, 0)),
          pl.BlockSpec((D_H, D_IN), lambda i: (0, 0)),
      ],
      out_specs=pl.BlockSpec((R, D_IN), lambda i: (i, 0)),
      out_shape=jax.ShapeDtypeStruct((NPAD, D_IN), jnp.float32),
  )(s, s, hp, deg2, deg2, b, W)


def kernel(x, edge_index, is_test, W1, b1, W2, b2, W3, b3):
  del is_test

  pad_idx = N + (jnp.arange(ETP - E, dtype=jnp.int32) % PAD_ROWS)
  srcp = jnp.concatenate(
      [edge_index[0].astype(jnp.int32), pad_idx]).reshape(IDX_ROWS, CHUNK)
  dstp = jnp.concatenate(
      [edge_index[1].astype(jnp.int32), pad_idx]).reshape(IDX_ROWS, CHUNK)
  xp = jnp.pad(x, ((0, NPAD - N), (0, 0)))
  b1r = b1.reshape(1, D_H)
  b2r = b2.reshape(1, D_H)
  b3r = b3.reshape(1, D_IN)

  deg2 = _deg_kernel(dstp)
  h1p = _k1(xp, W1, deg2)
  s1 = _agg_kernel(srcp, dstp, h1p)
  h2p = _k2(s1, h1p, deg2, b1r, W2)
  s2 = _agg_kernel(srcp, dstp, h2p)
  h3p = _k2b(s2, h2p, deg2, b2r)
  s3 = _agg_kernel(srcp, dstp, h3p)
  out = _k3(s3, h3p, deg2, b3r, W3)
  return out[:N]
